# baseline (speedup 1.0000x reference)
import jax
import jax.numpy as jnp
from jax.experimental import pallas as pl
from jax.experimental.pallas import tpu as pltpu

M = 1_000_000
D = 64
WT = 8192


def _tc_body(u_ref, x_ref, o_ref):
    x = x_ref[...]
    u = u_ref[...]
    o_ref[...] = jnp.sum(x * u, axis=0)


@jax.jit
def _tc_matvec(items_t, u_col):
    grid = ((M + WT - 1) // WT,)
    return pl.pallas_call(
        _tc_body,
        grid=grid,
        in_specs=[
            pl.BlockSpec((D, 1), lambda i: (0, 0)),
            pl.BlockSpec((D, WT), lambda i: (0, i)),
        ],
        out_specs=pl.BlockSpec((WT,), lambda i: (i,)),
        out_shape=jax.ShapeDtypeStruct((M,), jnp.float32),
    )(u_col, items_t)


def kernel(items_emb, user_emb):
    return _tc_matvec(items_emb.T, user_emb.reshape(D, 1))


# speedup vs baseline: 1.5940x; 1.5940x over previous
import jax
import jax.numpy as jnp
from jax.experimental import pallas as pl
from jax.experimental.pallas import tpu as pltpu

M = 1_000_000
D = 64
WT = 32768


def _tc_body(u_ref, x_ref, o_ref):
    x = x_ref[...]
    u = u_ref[...]
    o_ref[...] = jnp.sum(x * u, axis=0)


@jax.jit
def _tc_matvec(items_t, u_col):
    grid = ((M + WT - 1) // WT,)
    return pl.pallas_call(
        _tc_body,
        grid=grid,
        in_specs=[
            pl.BlockSpec((D, 1), lambda i: (0, 0)),
            pl.BlockSpec((D, WT), lambda i: (0, i)),
        ],
        out_specs=pl.BlockSpec((WT,), lambda i: (i,)),
        out_shape=jax.ShapeDtypeStruct((M,), jnp.float32),
    )(u_col, items_t)


def kernel(items_emb, user_emb):
    return _tc_matvec(items_emb.T, user_emb.reshape(D, 1))
